# baseline (speedup 1.0000x reference)
import math

import jax
import jax.numpy as jnp
from jax.experimental import pallas as pl
from jax.experimental.pallas import tpu as pltpu

_B, _T, _C = 4, 2048, 1024
_E, _K = 8, 2
_F = 4 * _C
_BF = 512
_NF = _F // _BF

_GELU_C = math.sqrt(2.0 / math.pi)


def _gelu(v):
    return 0.5 * v * (1.0 + jnp.tanh(_GELU_C * (v + 0.044715 * v * v * v)))


def _router_kernel(x_ref, rw_ref, sel_ref, wts_ref, loss_ref, logits_scr):
    b = pl.program_id(0)
    mean = jnp.mean(x_ref[0], axis=0, keepdims=True)
    logits = jax.lax.dot_general(
        mean, rw_ref[...], (((1,), (1,)), ((), ())),
        preferred_element_type=jnp.float32)
    logits_scr[pl.ds(b, 1), :] = logits

    @pl.when(b == _B - 1)
    def _finalize():
        lg = logits_scr[...]
        m = jnp.max(lg, axis=-1, keepdims=True)
        p = jnp.exp(lg - m)
        probs = p / jnp.sum(p, axis=-1, keepdims=True)
        iota = jax.lax.broadcasted_iota(jnp.int32, (_B, _E), 1)
        m1 = jnp.max(probs, axis=-1, keepdims=True)
        i1 = jnp.min(jnp.where(probs == m1, iota, _E), axis=-1, keepdims=True)
        masked = jnp.where(iota == i1, -jnp.inf, probs)
        m2 = jnp.max(masked, axis=-1, keepdims=True)
        i2 = jnp.min(jnp.where(masked == m2, iota, _E), axis=-1, keepdims=True)
        tot = m1 + m2
        sel_ref[...] = jnp.concatenate([i1, i2], axis=-1)
        wts_ref[...] = jnp.concatenate([m1 / tot, m2 / tot], axis=-1)
        importance = jnp.sum(probs, axis=0, keepdims=True) / _B
        assigned = ((iota == i1) | (iota == i2)).astype(jnp.float32)
        load = jnp.sum(assigned, axis=0, keepdims=True) / _B
        loss_ref[...] = jnp.reshape(_E * jnp.sum(importance * load), (1, 1))


def _moe_kernel(sel_ref, wts_ref, x_ref, w1_ref, b1_ref, w2_ref, b2_ref,
                out_ref):
    b = pl.program_id(0)
    k = pl.program_id(1)
    f = pl.program_id(2)
    h = jax.lax.dot_general(
        x_ref[0].astype(jnp.bfloat16), w1_ref[0].astype(jnp.bfloat16),
        (((1,), (1,)), ((), ())),
        preferred_element_type=jnp.float32)
    h = _gelu(h + b1_ref[0])
    o = jax.lax.dot_general(
        h.astype(jnp.bfloat16), w2_ref[0].astype(jnp.bfloat16),
        (((1,), (1,)), ((), ())),
        preferred_element_type=jnp.float32)
    w = wts_ref[b, k]
    first_f = (f == 0).astype(jnp.float32)
    contrib = w * (o + first_f * b2_ref[0])

    @pl.when((k == 0) & (f == 0))
    def _init():
        out_ref[0] = contrib

    @pl.when((k > 0) | (f > 0))
    def _accum():
        out_ref[0] += contrib


def kernel(x, router_w, c_fc_w, c_fc_b, c_proj_w, c_proj_b):
    sel, wts, loss = pl.pallas_call(
        _router_kernel,
        grid=(_B,),
        in_specs=[
            pl.BlockSpec((1, _T, _C), lambda b: (b, 0, 0)),
            pl.BlockSpec((_E, _C), lambda b: (0, 0)),
        ],
        out_specs=[
            pl.BlockSpec((_B, _K), lambda b: (0, 0)),
            pl.BlockSpec((_B, _K), lambda b: (0, 0)),
            pl.BlockSpec((1, 1), lambda b: (0, 0)),
        ],
        out_shape=[
            jax.ShapeDtypeStruct((_B, _K), jnp.int32),
            jax.ShapeDtypeStruct((_B, _K), jnp.float32),
            jax.ShapeDtypeStruct((1, 1), jnp.float32),
        ],
        scratch_shapes=[pltpu.VMEM((_B, _E), jnp.float32)],
    )(x, router_w)

    grid_spec = pltpu.PrefetchScalarGridSpec(
        num_scalar_prefetch=2,
        grid=(_B, _K, _NF),
        in_specs=[
            pl.BlockSpec((1, _T, _C), lambda b, k, f, s, w: (b, 0, 0)),
            pl.BlockSpec((1, _BF, _C), lambda b, k, f, s, w: (s[b, k], f, 0)),
            pl.BlockSpec((1, 1, _BF), lambda b, k, f, s, w: (s[b, k], 0, f)),
            pl.BlockSpec((1, _C, _BF), lambda b, k, f, s, w: (s[b, k], 0, f)),
            pl.BlockSpec((1, 1, _C), lambda b, k, f, s, w: (s[b, k], 0, 0)),
        ],
        out_specs=pl.BlockSpec((1, _T, _C), lambda b, k, f, s, w: (b, 0, 0)),
    )
    out = pl.pallas_call(
        _moe_kernel,
        grid_spec=grid_spec,
        out_shape=jax.ShapeDtypeStruct((_B, _T, _C), jnp.float32),
    )(sel, wts, x, c_fc_w, c_fc_b.reshape(_E, 1, _F), c_proj_w,
      c_proj_b.reshape(_E, 1, _C))
    return out, loss.reshape(())


# speedup vs baseline: 1.0464x; 1.0464x over previous
import math

import jax
import jax.numpy as jnp
from jax.experimental import pallas as pl
from jax.experimental.pallas import tpu as pltpu

_B, _T, _C = 4, 2048, 1024
_E, _K = 8, 2
_F = 4 * _C
_BF = 512
_NF = _F // _BF

_GELU_C = math.sqrt(2.0 / math.pi)


def _gelu(v):
    u = v * (_GELU_C + (_GELU_C * 0.044715) * (v * v))
    return (0.5 * v) * (1.0 + jnp.tanh(u))


def _router_kernel(x_ref, rw_ref, sel_ref, wts_ref, loss_ref, logits_scr):
    b = pl.program_id(0)
    mean = jnp.mean(x_ref[0], axis=0, keepdims=True)
    logits = jax.lax.dot_general(
        mean, rw_ref[...], (((1,), (1,)), ((), ())),
        preferred_element_type=jnp.float32)
    logits_scr[pl.ds(b, 1), :] = logits

    @pl.when(b == _B - 1)
    def _finalize():
        lg = logits_scr[...]
        m = jnp.max(lg, axis=-1, keepdims=True)
        p = jnp.exp(lg - m)
        probs = p / jnp.sum(p, axis=-1, keepdims=True)
        iota = jax.lax.broadcasted_iota(jnp.int32, (_B, _E), 1)
        m1 = jnp.max(probs, axis=-1, keepdims=True)
        i1 = jnp.min(jnp.where(probs == m1, iota, _E), axis=-1, keepdims=True)
        masked = jnp.where(iota == i1, -jnp.inf, probs)
        m2 = jnp.max(masked, axis=-1, keepdims=True)
        i2 = jnp.min(jnp.where(masked == m2, iota, _E), axis=-1, keepdims=True)
        tot = m1 + m2
        sel_ref[...] = jnp.concatenate([i1, i2], axis=-1)
        wts_ref[...] = jnp.concatenate([m1 / tot, m2 / tot], axis=-1)
        importance = jnp.sum(probs, axis=0, keepdims=True) / _B
        assigned = ((iota == i1) | (iota == i2)).astype(jnp.float32)
        load = jnp.sum(assigned, axis=0, keepdims=True) / _B
        loss_ref[...] = jnp.reshape(_E * jnp.sum(importance * load), (1, 1))


def _moe_kernel(sel_ref, wts_ref, x_ref, w1_ref, b1_ref, w2_ref, b2_ref,
                out_ref):
    b = pl.program_id(0)
    k = pl.program_id(1)
    f = pl.program_id(2)
    w = wts_ref[b, k]
    h = jax.lax.dot_general(
        x_ref[0], w1_ref[0], (((1,), (1,)), ((), ())),
        preferred_element_type=jnp.float32)
    h = _gelu(h + b1_ref[0])
    o = jax.lax.dot_general(
        h, w * w2_ref[0], (((1,), (1,)), ((), ())),
        preferred_element_type=jnp.float32)

    @pl.when((k == 0) & (f == 0))
    def _init():
        out_ref[0] = o + w * b2_ref[0]

    @pl.when((k > 0) & (f == 0))
    def _accum_bias():
        out_ref[0] += o + w * b2_ref[0]

    @pl.when(f > 0)
    def _accum():
        out_ref[0] += o


def kernel(x, router_w, c_fc_w, c_fc_b, c_proj_w, c_proj_b):
    sel, wts, loss = pl.pallas_call(
        _router_kernel,
        grid=(_B,),
        in_specs=[
            pl.BlockSpec((1, _T, _C), lambda b: (b, 0, 0)),
            pl.BlockSpec((_E, _C), lambda b: (0, 0)),
        ],
        out_specs=[
            pl.BlockSpec((_B, _K), lambda b: (0, 0)),
            pl.BlockSpec((_B, _K), lambda b: (0, 0)),
            pl.BlockSpec((1, 1), lambda b: (0, 0)),
        ],
        out_shape=[
            jax.ShapeDtypeStruct((_B, _K), jnp.int32),
            jax.ShapeDtypeStruct((_B, _K), jnp.float32),
            jax.ShapeDtypeStruct((1, 1), jnp.float32),
        ],
        scratch_shapes=[pltpu.VMEM((_B, _E), jnp.float32)],
    )(x, router_w)

    grid_spec = pltpu.PrefetchScalarGridSpec(
        num_scalar_prefetch=2,
        grid=(_B, _K, _NF),
        in_specs=[
            pl.BlockSpec((1, _T, _C), lambda b, k, f, s, w: (b, 0, 0)),
            pl.BlockSpec((1, _BF, _C), lambda b, k, f, s, w: (s[b, k], f, 0)),
            pl.BlockSpec((1, 1, _BF), lambda b, k, f, s, w: (s[b, k], 0, f)),
            pl.BlockSpec((1, _C, _BF), lambda b, k, f, s, w: (s[b, k], 0, f)),
            pl.BlockSpec((1, 1, _C), lambda b, k, f, s, w: (s[b, k], 0, 0)),
        ],
        out_specs=pl.BlockSpec((1, _T, _C), lambda b, k, f, s, w: (b, 0, 0)),
    )
    out = pl.pallas_call(
        _moe_kernel,
        grid_spec=grid_spec,
        out_shape=jax.ShapeDtypeStruct((_B, _T, _C), jnp.float32),
    )(sel, wts, x, c_fc_w, c_fc_b.reshape(_E, 1, _F), c_proj_w,
      c_proj_b.reshape(_E, 1, _C))
    return out, loss.reshape(())


# speedup vs baseline: 1.0469x; 1.0005x over previous
import math

import jax
import jax.numpy as jnp
from jax.experimental import pallas as pl
from jax.experimental.pallas import tpu as pltpu

_B, _T, _C = 4, 2048, 1024
_E, _K = 8, 2
_F = 4 * _C
_BF = 512
_NF = _F // _BF

_GELU_C = math.sqrt(2.0 / math.pi)


def _gelu(v):
    u = v * (_GELU_C + (_GELU_C * 0.044715) * (v * v))
    return (0.5 * v) * (1.0 + jnp.tanh(u))


def _router_kernel(x_ref, rw_ref, sel_ref, wts_ref, loss_ref, logits_scr):
    b = pl.program_id(0)
    mean = jnp.mean(x_ref[0], axis=0, keepdims=True)
    logits = jax.lax.dot_general(
        mean, rw_ref[...], (((1,), (1,)), ((), ())),
        preferred_element_type=jnp.float32)
    logits_scr[pl.ds(b, 1), :] = logits

    @pl.when(b == _B - 1)
    def _finalize():
        lg = logits_scr[...]
        m = jnp.max(lg, axis=-1, keepdims=True)
        p = jnp.exp(lg - m)
        probs = p / jnp.sum(p, axis=-1, keepdims=True)
        iota = jax.lax.broadcasted_iota(jnp.int32, (_B, _E), 1)
        m1 = jnp.max(probs, axis=-1, keepdims=True)
        i1 = jnp.min(jnp.where(probs == m1, iota, _E), axis=-1, keepdims=True)
        masked = jnp.where(iota == i1, -jnp.inf, probs)
        m2 = jnp.max(masked, axis=-1, keepdims=True)
        i2 = jnp.min(jnp.where(masked == m2, iota, _E), axis=-1, keepdims=True)
        tot = m1 + m2
        sel_ref[...] = jnp.concatenate([i1, i2], axis=-1)
        wts_ref[...] = jnp.concatenate([m1 / tot, m2 / tot], axis=-1)
        importance = jnp.sum(probs, axis=0, keepdims=True) / _B
        assigned = ((iota == i1) | (iota == i2)).astype(jnp.float32)
        load = jnp.sum(assigned, axis=0, keepdims=True) / _B
        loss_ref[...] = jnp.reshape(_E * jnp.sum(importance * load), (1, 1))


def _moe_kernel(sel_ref, wts_ref, x_ref, w1_ref, b1_ref, w2_ref, b2_ref,
                out_ref):
    b = pl.program_id(0)
    k = pl.program_id(1)
    f = pl.program_id(2)
    w = wts_ref[b, k]
    h = jax.lax.dot_general(
        x_ref[0], w1_ref[0], (((1,), (1,)), ((), ())),
        preferred_element_type=jnp.float32)
    h = _gelu(h + b1_ref[0])
    o = jax.lax.dot_general(
        h, w * w2_ref[0], (((1,), (1,)), ((), ())),
        preferred_element_type=jnp.float32)

    @pl.when((k == 0) & (f == 0))
    def _init():
        out_ref[0] = o + w * b2_ref[0]

    @pl.when((k > 0) & (f == 0))
    def _accum_bias():
        out_ref[0] += o + w * b2_ref[0]

    @pl.when(f > 0)
    def _accum():
        out_ref[0] += o


def kernel(x, router_w, c_fc_w, c_fc_b, c_proj_w, c_proj_b):
    sel, wts, loss = pl.pallas_call(
        _router_kernel,
        grid=(_B,),
        in_specs=[
            pl.BlockSpec((1, _T, _C), lambda b: (b, 0, 0)),
            pl.BlockSpec((_E, _C), lambda b: (0, 0)),
        ],
        out_specs=[
            pl.BlockSpec((_B, _K), lambda b: (0, 0)),
            pl.BlockSpec((_B, _K), lambda b: (0, 0)),
            pl.BlockSpec((1, 1), lambda b: (0, 0)),
        ],
        out_shape=[
            jax.ShapeDtypeStruct((_B, _K), jnp.int32),
            jax.ShapeDtypeStruct((_B, _K), jnp.float32),
            jax.ShapeDtypeStruct((1, 1), jnp.float32),
        ],
        scratch_shapes=[pltpu.VMEM((_B, _E), jnp.float32)],
    )(x, router_w)

    grid_spec = pltpu.PrefetchScalarGridSpec(
        num_scalar_prefetch=2,
        grid=(_B, _K, _NF),
        in_specs=[
            pl.BlockSpec((1, _T, _C), lambda b, k, f, s, w: (b, 0, 0)),
            pl.BlockSpec((1, _BF, _C), lambda b, k, f, s, w: (s[b, k], f, 0)),
            pl.BlockSpec((1, 1, _BF), lambda b, k, f, s, w: (s[b, k], 0, f)),
            pl.BlockSpec((1, _C, _BF), lambda b, k, f, s, w: (s[b, k], 0, f)),
            pl.BlockSpec((1, 1, _C), lambda b, k, f, s, w: (s[b, k], 0, 0)),
        ],
        out_specs=pl.BlockSpec((1, _T, _C), lambda b, k, f, s, w: (b, 0, 0)),
    )
    out = pl.pallas_call(
        _moe_kernel,
        grid_spec=grid_spec,
        out_shape=jax.ShapeDtypeStruct((_B, _T, _C), jnp.float32),
        compiler_params=pltpu.CompilerParams(
            dimension_semantics=("parallel", "arbitrary", "arbitrary")),
    )(sel, wts, x, c_fc_w, c_fc_b.reshape(_E, 1, _F), c_proj_w,
      c_proj_b.reshape(_E, 1, _C))
    return out, loss.reshape(())


# speedup vs baseline: 1.1254x; 1.0750x over previous
import math

import jax
import jax.numpy as jnp
from jax.experimental import pallas as pl
from jax.experimental.pallas import tpu as pltpu

_B, _T, _C = 4, 2048, 1024
_E, _K = 8, 2
_F = 4 * _C
_BF = 1024
_NF = _F // _BF

_GELU_C = math.sqrt(2.0 / math.pi)


def _gelu(v):
    u = v * (_GELU_C + (_GELU_C * 0.044715) * (v * v))
    return (0.5 * v) * (1.0 + jnp.tanh(u))


def _router_kernel(x_ref, rw_ref, sel_ref, wts_ref, loss_ref, logits_scr):
    b = pl.program_id(0)
    mean = jnp.mean(x_ref[0], axis=0, keepdims=True)
    logits = jax.lax.dot_general(
        mean, rw_ref[...], (((1,), (1,)), ((), ())),
        preferred_element_type=jnp.float32)
    logits_scr[pl.ds(b, 1), :] = logits

    @pl.when(b == _B - 1)
    def _finalize():
        lg = logits_scr[...]
        m = jnp.max(lg, axis=-1, keepdims=True)
        p = jnp.exp(lg - m)
        probs = p / jnp.sum(p, axis=-1, keepdims=True)
        iota = jax.lax.broadcasted_iota(jnp.int32, (_B, _E), 1)
        m1 = jnp.max(probs, axis=-1, keepdims=True)
        i1 = jnp.min(jnp.where(probs == m1, iota, _E), axis=-1, keepdims=True)
        masked = jnp.where(iota == i1, -jnp.inf, probs)
        m2 = jnp.max(masked, axis=-1, keepdims=True)
        i2 = jnp.min(jnp.where(masked == m2, iota, _E), axis=-1, keepdims=True)
        tot = m1 + m2
        sel_ref[...] = jnp.concatenate([i1, i2], axis=-1)
        wts_ref[...] = jnp.concatenate([m1 / tot, m2 / tot], axis=-1)
        importance = jnp.sum(probs, axis=0, keepdims=True) / _B
        assigned = ((iota == i1) | (iota == i2)).astype(jnp.float32)
        load = jnp.sum(assigned, axis=0, keepdims=True) / _B
        loss_ref[...] = jnp.reshape(_E * jnp.sum(importance * load), (1, 1))


def _moe_kernel(sel_ref, wts_ref, x_ref, w1_ref, b1_ref, w2_ref, b2_ref,
                out_ref):
    b = pl.program_id(0)
    k = pl.program_id(1)
    f = pl.program_id(2)
    w = wts_ref[b, k]
    h = jax.lax.dot_general(
        x_ref[0], w1_ref[0], (((1,), (1,)), ((), ())),
        preferred_element_type=jnp.float32)
    h = _gelu(h + b1_ref[0])
    o = jax.lax.dot_general(
        h, w * w2_ref[0], (((1,), (1,)), ((), ())),
        preferred_element_type=jnp.float32)

    @pl.when((k == 0) & (f == 0))
    def _init():
        out_ref[0] = o + w * b2_ref[0]

    @pl.when((k > 0) & (f == 0))
    def _accum_bias():
        out_ref[0] += o + w * b2_ref[0]

    @pl.when(f > 0)
    def _accum():
        out_ref[0] += o


def kernel(x, router_w, c_fc_w, c_fc_b, c_proj_w, c_proj_b):
    sel, wts, loss = pl.pallas_call(
        _router_kernel,
        grid=(_B,),
        in_specs=[
            pl.BlockSpec((1, _T, _C), lambda b: (b, 0, 0)),
            pl.BlockSpec((_E, _C), lambda b: (0, 0)),
        ],
        out_specs=[
            pl.BlockSpec((_B, _K), lambda b: (0, 0)),
            pl.BlockSpec((_B, _K), lambda b: (0, 0)),
            pl.BlockSpec((1, 1), lambda b: (0, 0)),
        ],
        out_shape=[
            jax.ShapeDtypeStruct((_B, _K), jnp.int32),
            jax.ShapeDtypeStruct((_B, _K), jnp.float32),
            jax.ShapeDtypeStruct((1, 1), jnp.float32),
        ],
        scratch_shapes=[pltpu.VMEM((_B, _E), jnp.float32)],
    )(x, router_w)

    grid_spec = pltpu.PrefetchScalarGridSpec(
        num_scalar_prefetch=2,
        grid=(_B, _K, _NF),
        in_specs=[
            pl.BlockSpec((1, _T, _C), lambda b, k, f, s, w: (b, 0, 0)),
            pl.BlockSpec((1, _BF, _C), lambda b, k, f, s, w: (s[b, k], f, 0)),
            pl.BlockSpec((1, 1, _BF), lambda b, k, f, s, w: (s[b, k], 0, f)),
            pl.BlockSpec((1, _C, _BF), lambda b, k, f, s, w: (s[b, k], 0, f)),
            pl.BlockSpec((1, 1, _C), lambda b, k, f, s, w: (s[b, k], 0, 0)),
        ],
        out_specs=pl.BlockSpec((1, _T, _C), lambda b, k, f, s, w: (b, 0, 0)),
    )
    out = pl.pallas_call(
        _moe_kernel,
        grid_spec=grid_spec,
        out_shape=jax.ShapeDtypeStruct((_B, _T, _C), jnp.float32),
        compiler_params=pltpu.CompilerParams(
            dimension_semantics=("parallel", "arbitrary", "arbitrary"),
            vmem_limit_bytes=100 * 1024 * 1024),
    )(sel, wts, x, c_fc_w, c_fc_b.reshape(_E, 1, _F), c_proj_w,
      c_proj_b.reshape(_E, 1, _C))
    return out, loss.reshape(())


# speedup vs baseline: 1.1303x; 1.0044x over previous
import math

import jax
import jax.numpy as jnp
from jax.experimental import pallas as pl
from jax.experimental.pallas import tpu as pltpu

_B, _T, _C = 4, 2048, 1024
_E, _K = 8, 2
_F = 4 * _C
_BF = 1024
_NF = _F // _BF

_GELU_C = math.sqrt(2.0 / math.pi)


def _gelu(v):
    u = v * (_GELU_C + (_GELU_C * 0.044715) * (v * v))
    return (0.5 * v) * (1.0 + jnp.tanh(u))


def _router_kernel(x_ref, rw_ref, sel_ref, wts_ref, loss_ref, logits_scr):
    b = pl.program_id(0)
    mean = jnp.mean(x_ref[0], axis=0, keepdims=True)
    logits = jax.lax.dot_general(
        mean, rw_ref[...], (((1,), (1,)), ((), ())),
        preferred_element_type=jnp.float32)
    logits_scr[pl.ds(b, 1), :] = logits

    @pl.when(b == _B - 1)
    def _finalize():
        lg = logits_scr[...]
        m = jnp.max(lg, axis=-1, keepdims=True)
        p = jnp.exp(lg - m)
        probs = p / jnp.sum(p, axis=-1, keepdims=True)
        iota = jax.lax.broadcasted_iota(jnp.int32, (_B, _E), 1)
        m1 = jnp.max(probs, axis=-1, keepdims=True)
        i1 = jnp.min(jnp.where(probs == m1, iota, _E), axis=-1, keepdims=True)
        masked = jnp.where(iota == i1, -jnp.inf, probs)
        m2 = jnp.max(masked, axis=-1, keepdims=True)
        i2 = jnp.min(jnp.where(masked == m2, iota, _E), axis=-1, keepdims=True)
        tot = m1 + m2
        sel_ref[...] = jnp.concatenate([i1, i2], axis=-1)
        wts_ref[...] = jnp.concatenate([m1 / tot, m2 / tot], axis=-1)
        importance = jnp.sum(probs, axis=0, keepdims=True) / _B
        assigned = ((iota == i1) | (iota == i2)).astype(jnp.float32)
        load = jnp.sum(assigned, axis=0, keepdims=True) / _B
        loss_ref[...] = jnp.reshape(_E * jnp.sum(importance * load), (1, 1))


def _moe_kernel(sel_ref, wts_ref, x_ref, w1_ref, b1_ref, w2_ref, b2_ref,
                out_ref):
    b = pl.program_id(0)
    k = pl.program_id(1)
    f = pl.program_id(2)
    w = wts_ref[b, k]
    h32 = jax.lax.dot_general(
        x_ref[0], w1_ref[0].astype(jnp.bfloat16), (((1,), (1,)), ((), ())),
        preferred_element_type=jnp.float32)
    h = _gelu(h32.astype(jnp.bfloat16) + b1_ref[0].astype(jnp.bfloat16))
    o = jax.lax.dot_general(
        h, (w * w2_ref[0]).astype(jnp.bfloat16), (((1,), (1,)), ((), ())),
        preferred_element_type=jnp.float32)

    @pl.when((k == 0) & (f == 0))
    def _init():
        out_ref[0] = o + w * b2_ref[0]

    @pl.when((k > 0) & (f == 0))
    def _accum_bias():
        out_ref[0] += o + w * b2_ref[0]

    @pl.when(f > 0)
    def _accum():
        out_ref[0] += o


def kernel(x, router_w, c_fc_w, c_fc_b, c_proj_w, c_proj_b):
    sel, wts, loss = pl.pallas_call(
        _router_kernel,
        grid=(_B,),
        in_specs=[
            pl.BlockSpec((1, _T, _C), lambda b: (b, 0, 0)),
            pl.BlockSpec((_E, _C), lambda b: (0, 0)),
        ],
        out_specs=[
            pl.BlockSpec((_B, _K), lambda b: (0, 0)),
            pl.BlockSpec((_B, _K), lambda b: (0, 0)),
            pl.BlockSpec((1, 1), lambda b: (0, 0)),
        ],
        out_shape=[
            jax.ShapeDtypeStruct((_B, _K), jnp.int32),
            jax.ShapeDtypeStruct((_B, _K), jnp.float32),
            jax.ShapeDtypeStruct((1, 1), jnp.float32),
        ],
        scratch_shapes=[pltpu.VMEM((_B, _E), jnp.float32)],
    )(x, router_w)

    grid_spec = pltpu.PrefetchScalarGridSpec(
        num_scalar_prefetch=2,
        grid=(_B, _K, _NF),
        in_specs=[
            pl.BlockSpec((1, _T, _C), lambda b, k, f, s, w: (b, 0, 0)),
            pl.BlockSpec((1, _BF, _C), lambda b, k, f, s, w: (s[b, k], f, 0)),
            pl.BlockSpec((1, 1, _BF), lambda b, k, f, s, w: (s[b, k], 0, f)),
            pl.BlockSpec((1, _C, _BF), lambda b, k, f, s, w: (s[b, k], 0, f)),
            pl.BlockSpec((1, 1, _C), lambda b, k, f, s, w: (s[b, k], 0, 0)),
        ],
        out_specs=pl.BlockSpec((1, _T, _C), lambda b, k, f, s, w: (b, 0, 0)),
    )
    out = pl.pallas_call(
        _moe_kernel,
        grid_spec=grid_spec,
        out_shape=jax.ShapeDtypeStruct((_B, _T, _C), jnp.float32),
        compiler_params=pltpu.CompilerParams(
            dimension_semantics=("parallel", "arbitrary", "arbitrary"),
            vmem_limit_bytes=100 * 1024 * 1024),
    )(sel, wts, x.astype(jnp.bfloat16), c_fc_w, c_fc_b.reshape(_E, 1, _F),
      c_proj_w, c_proj_b.reshape(_E, 1, _C))
    return out, loss.reshape(())
